# Initial kernel scaffold; baseline (speedup 1.0000x reference)
import jax
import jax.numpy as jnp
from jax.experimental import pallas as pl


def kernel(h, edge_index, edge_attr, lin_edge_W, lin_edge_b, mlp_W1, mlp_b1, mlp_W2, mlp_b2, eps, bn_gamma, bn_beta):
    raise NotImplementedError("write your pallas kernel here")


# speedup vs baseline: 2.5712x; 2.5712x over previous
import functools

import jax
import jax.numpy as jnp
from jax import lax
from jax.experimental import pallas as pl
from jax.experimental.pallas import tpu as pltpu
from jax.experimental.pallas import tpu_sc as plsc

N = 10000
E = 320000
D = 128
DE = 16

NC = 2
NS = 16
NW = NC * NS
EW = E // NW
C = 80
NCHUNK = EW // C
NPAD = 10240
ROWS_PER_TILE = NPAD // NS
ZROWS = 128


def _emb_body(attr_ref, wt_ref, b_ref, out_ref):
    out_ref[...] = (
        jnp.dot(attr_ref[...], wt_ref[...], preferred_element_type=jnp.float32)
        + b_ref[...]
    )


def _edge_emb(edge_attr, wt, b2d):
    BE = 4000
    return pl.pallas_call(
        _emb_body,
        grid=(E // BE,),
        in_specs=[
            pl.BlockSpec((BE, DE), lambda i: (i, 0)),
            pl.BlockSpec((DE, D), lambda i: (0, 0)),
            pl.BlockSpec((1, D), lambda i: (0, 0)),
        ],
        out_specs=pl.BlockSpec((BE, D), lambda i: (i, 0)),
        out_shape=jax.ShapeDtypeStruct((E, D), jnp.float32),
    )(edge_attr, wt, b2d)


def _sc_kernel_body(src_hbm, dst_hbm, emb_hbm, h_hbm, out_hbm,
                    src_v, dst_v, emb_v, rows_v, stage_v, aggr_sh, sem):
    cid = lax.axis_index("c")
    sid = lax.axis_index("s")
    wid = sid * NC + cid

    def zrow(r, carry):
        for j in range(D // 16):
            stage_v[r, pl.ds(j * 16, 16)] = jnp.zeros((16,), jnp.float32)
        return carry

    lax.fori_loop(0, ZROWS, zrow, 0)
    row_base = sid * ROWS_PER_TILE
    for t in range(ROWS_PER_TILE // ZROWS):
        pltpu.sync_copy(stage_v, aggr_sh.at[pl.ds(row_base + t * ZROWS, ZROWS)])
    plsc.subcore_barrier()

    def chunk(i, carry):
        eb = wid * EW + i * C
        pltpu.sync_copy(src_hbm.at[pl.ds(eb, C)], src_v)
        pltpu.sync_copy(dst_hbm.at[pl.ds(eb, C)], dst_v)
        pltpu.sync_copy(emb_hbm.at[pl.ds(eb, C)], emb_v)
        pltpu.async_copy(h_hbm.at[src_v], rows_v, sem).wait()

        def edge(e, c2):
            for j in range(D // 16):
                s = pl.ds(j * 16, 16)
                rows_v[e, s] = jnp.maximum(rows_v[e, s] + emb_v[e, s], 0.0)
            return c2

        lax.fori_loop(0, C, edge, 0)
        pltpu.sync_copy(rows_v, aggr_sh.at[dst_v], add=True)
        return carry

    lax.fori_loop(0, NCHUNK, chunk, 0)
    plsc.subcore_barrier()

    for t in range(ROWS_PER_TILE // ZROWS):
        r0 = row_base + t * ZROWS
        pltpu.sync_copy(aggr_sh.at[pl.ds(r0, ZROWS)], stage_v)
        pltpu.sync_copy(stage_v, out_hbm.at[cid, pl.ds(r0, ZROWS)])


def _sc_aggregate(src, dst, emb, h):
    mesh = plsc.VectorSubcoreMesh(core_axis_name="c", subcore_axis_name="s")
    k = functools.partial(
        pl.kernel,
        mesh=mesh,
        out_type=jax.ShapeDtypeStruct((NC, NPAD, D), jnp.float32),
        scratch_types=[
            pltpu.VMEM((C,), jnp.int32),
            pltpu.VMEM((C,), jnp.int32),
            pltpu.VMEM((C, D), jnp.float32),
            pltpu.VMEM((C, D), jnp.float32),
            pltpu.VMEM((ZROWS, D), jnp.float32),
            pltpu.VMEM_SHARED((NPAD, D), jnp.float32),
            pltpu.SemaphoreType.DMA,
        ],
    )(_sc_kernel_body)
    return k(src, dst, emb, h)


def _epi_body(h_ref, a_ref, w1_ref, b1_ref, w2_ref, b2_ref, eps_ref, g_ref,
              bt_ref, out_ref):
    h = h_ref[...]
    x = (1.0 + eps_ref[0, 0]) * h + a_ref[0, :N] + a_ref[1, :N]
    y = jnp.maximum(
        jnp.dot(x, w1_ref[...], preferred_element_type=jnp.float32) + b1_ref[...],
        0.0,
    )
    y = jnp.dot(y, w2_ref[...], preferred_element_type=jnp.float32) + b2_ref[...]
    mean = jnp.mean(y, axis=0, keepdims=True)
    var = jnp.mean((y - mean) ** 2, axis=0, keepdims=True)
    out_ref[...] = g_ref[...] * (y - mean) * lax.rsqrt(var + 1e-5) + bt_ref[...] + h


def _epilogue(h, aggr, w1t, b1, w2t, b2, eps, gamma, beta):
    return pl.pallas_call(
        _epi_body,
        out_shape=jax.ShapeDtypeStruct((N, D), jnp.float32),
    )(h, aggr, w1t, b1.reshape(1, D), w2t, b2.reshape(1, D),
      eps.reshape(1, 1), gamma.reshape(1, D), beta.reshape(1, D))


def kernel(h, edge_index, edge_attr, lin_edge_W, lin_edge_b, mlp_W1, mlp_b1,
           mlp_W2, mlp_b2, eps, bn_gamma, bn_beta):
    src = edge_index[0].astype(jnp.int32)
    dst = edge_index[1].astype(jnp.int32)
    emb = _edge_emb(edge_attr, lin_edge_W.T, lin_edge_b.reshape(1, D))
    aggr = _sc_aggregate(src, dst, emb, h)
    return _epilogue(h, aggr, mlp_W1.T, mlp_b1, mlp_W2.T, mlp_b2, eps,
                     bn_gamma, bn_beta)


# speedup vs baseline: 3.3821x; 1.3153x over previous
import functools

import jax
import jax.numpy as jnp
from jax import lax
from jax.experimental import pallas as pl
from jax.experimental.pallas import tpu as pltpu
from jax.experimental.pallas import tpu_sc as plsc

N = 10000
E = 320000
D = 128
DE = 16

NC = 2
NS = 16
NW = NC * NS
EW = E // NW
C = 40
NCHUNK = EW // C
NBUF = 4
NPAD = 10240
ROWS_PER_TILE = NPAD // NS
ZROWS = 40


def _emb_body(attr_ref, wt_ref, b_ref, out_ref):
    out_ref[...] = (
        jnp.dot(attr_ref[...], wt_ref[...], preferred_element_type=jnp.float32)
        + b_ref[...]
    )


def _edge_emb(edge_attr, wt, b2d):
    BE = 4000
    return pl.pallas_call(
        _emb_body,
        grid=(E // BE,),
        in_specs=[
            pl.BlockSpec((BE, DE), lambda i: (i, 0)),
            pl.BlockSpec((DE, D), lambda i: (0, 0)),
            pl.BlockSpec((1, D), lambda i: (0, 0)),
        ],
        out_specs=pl.BlockSpec((BE, D), lambda i: (i, 0)),
        out_shape=jax.ShapeDtypeStruct((E, D), jnp.float32),
    )(edge_attr, wt, b2d)


def _sc_kernel_body(src_hbm, dst_hbm, emb_hbm, h_hbm, out_hbm,
                    src_v, dst_v, emb_v, rows_v, stage_v, aggr_sh,
                    sem_src, sem_dst, sem_emb, sem_gat, sem_sct):
    cid = lax.axis_index("c")
    sid = lax.axis_index("s")
    wid = sid * NC + cid
    ebase = wid * EW

    def zrow(r, carry):
        for j in range(D // 16):
            stage_v[r, pl.ds(j * 16, 16)] = jnp.zeros((16,), jnp.float32)
        return carry

    lax.fori_loop(0, ZROWS, zrow, 0)
    row_base = sid * ROWS_PER_TILE
    for t in range(ROWS_PER_TILE // ZROWS):
        pltpu.sync_copy(stage_v, aggr_sh.at[pl.ds(row_base + t * ZROWS, ZROWS)])
    plsc.subcore_barrier()

    def start_idx_emb(i, b):
        eb = ebase + i * C
        pltpu.async_copy(src_hbm.at[pl.ds(eb, C)], src_v.at[b], sem_src.at[b])
        pltpu.async_copy(dst_hbm.at[pl.ds(eb, C)], dst_v.at[b], sem_dst.at[b])
        pltpu.async_copy(emb_hbm.at[pl.ds(eb, C)], emb_v.at[b], sem_emb.at[b])

    def wait_idx(i, b):
        eb = ebase + i * C
        pltpu.make_async_copy(src_hbm.at[pl.ds(eb, C)], src_v.at[b],
                              sem_src.at[b]).wait()
        pltpu.make_async_copy(dst_hbm.at[pl.ds(eb, C)], dst_v.at[b],
                              sem_dst.at[b]).wait()

    def start_gather(b):
        pltpu.async_copy(h_hbm.at[src_v.at[b]], rows_v.at[b], sem_gat.at[b])

    def wait_sct(b):
        pltpu.make_async_copy(rows_v.at[b], aggr_sh.at[dst_v.at[b]],
                              sem_sct.at[b]).wait()

    def body(i, b, wait_scatter, do_prefetch, do_gather_next):
        b1 = (b + 1) % NBUF
        b2 = (b + 2) % NBUF
        eb = ebase + i * C
        pltpu.make_async_copy(emb_hbm.at[pl.ds(eb, C)], emb_v.at[b],
                              sem_emb.at[b]).wait()
        pltpu.make_async_copy(h_hbm.at[src_v.at[b]], rows_v.at[b],
                              sem_gat.at[b]).wait()

        def edge(e, c2):
            for j in range(D // 16):
                s = pl.ds(j * 16, 16)
                rows_v[b, e, s] = jnp.maximum(rows_v[b, e, s] + emb_v[b, e, s],
                                              0.0)
            return c2

        lax.fori_loop(0, C, edge, 0)
        pltpu.async_copy(rows_v.at[b], aggr_sh.at[dst_v.at[b]], sem_sct.at[b],
                         add=True)
        if wait_scatter:
            wait_sct(b2)
        if do_prefetch:
            start_idx_emb(i + 2, b2)
        if do_gather_next:
            wait_idx(i + 1, b1)
            start_gather(b1)

    start_idx_emb(0, 0)
    start_idx_emb(1, 1)
    wait_idx(0, 0)
    start_gather(0)
    body(0, 0, wait_scatter=False, do_prefetch=True, do_gather_next=True)
    body(1, 1, wait_scatter=False, do_prefetch=True, do_gather_next=True)
    body(2, 2, wait_scatter=True, do_prefetch=True, do_gather_next=True)
    body(3, 3, wait_scatter=True, do_prefetch=True, do_gather_next=True)

    def group(g, carry):
        i0 = g * NBUF
        for b in range(NBUF):
            body(i0 + b, b, wait_scatter=True, do_prefetch=True,
                 do_gather_next=True)
        return carry

    lax.fori_loop(1, (NCHUNK - 2) // NBUF, group, 0)

    body(NCHUNK - 2, 0, wait_scatter=True, do_prefetch=False,
         do_gather_next=True)
    body(NCHUNK - 1, 1, wait_scatter=True, do_prefetch=False,
         do_gather_next=False)
    wait_sct(0)
    wait_sct(1)
    plsc.subcore_barrier()

    for t in range(ROWS_PER_TILE // ZROWS):
        r0 = row_base + t * ZROWS
        pltpu.sync_copy(aggr_sh.at[pl.ds(r0, ZROWS)], stage_v)
        pltpu.sync_copy(stage_v, out_hbm.at[cid, pl.ds(r0, ZROWS)])


def _sc_aggregate(src, dst, emb, h):
    mesh = plsc.VectorSubcoreMesh(core_axis_name="c", subcore_axis_name="s")
    k = functools.partial(
        pl.kernel,
        mesh=mesh,
        out_type=jax.ShapeDtypeStruct((NC, NPAD, D), jnp.float32),
        scratch_types=[
            pltpu.VMEM((NBUF, C), jnp.int32),
            pltpu.VMEM((NBUF, C), jnp.int32),
            pltpu.VMEM((NBUF, C, D), jnp.float32),
            pltpu.VMEM((NBUF, C, D), jnp.float32),
            pltpu.VMEM((ZROWS, D), jnp.float32),
            pltpu.VMEM_SHARED((NPAD, D), jnp.float32),
            pltpu.SemaphoreType.DMA((NBUF,)),
            pltpu.SemaphoreType.DMA((NBUF,)),
            pltpu.SemaphoreType.DMA((NBUF,)),
            pltpu.SemaphoreType.DMA((NBUF,)),
            pltpu.SemaphoreType.DMA((NBUF,)),
        ],
    )(_sc_kernel_body)
    return k(src, dst, emb, h)


def _epi_body(h_ref, a_ref, w1_ref, b1_ref, w2_ref, b2_ref, eps_ref, g_ref,
              bt_ref, out_ref):
    h = h_ref[...]
    x = (1.0 + eps_ref[0, 0]) * h + a_ref[0, :N] + a_ref[1, :N]
    y = jnp.maximum(
        jnp.dot(x, w1_ref[...], preferred_element_type=jnp.float32) + b1_ref[...],
        0.0,
    )
    y = jnp.dot(y, w2_ref[...], preferred_element_type=jnp.float32) + b2_ref[...]
    mean = jnp.mean(y, axis=0, keepdims=True)
    var = jnp.mean((y - mean) ** 2, axis=0, keepdims=True)
    out_ref[...] = g_ref[...] * (y - mean) * lax.rsqrt(var + 1e-5) + bt_ref[...] + h


def _epilogue(h, aggr, w1t, b1, w2t, b2, eps, gamma, beta):
    return pl.pallas_call(
        _epi_body,
        out_shape=jax.ShapeDtypeStruct((N, D), jnp.float32),
    )(h, aggr, w1t, b1.reshape(1, D), w2t, b2.reshape(1, D),
      eps.reshape(1, 1), gamma.reshape(1, D), beta.reshape(1, D))


def kernel(h, edge_index, edge_attr, lin_edge_W, lin_edge_b, mlp_W1, mlp_b1,
           mlp_W2, mlp_b2, eps, bn_gamma, bn_beta):
    src = edge_index[0].astype(jnp.int32)
    dst = edge_index[1].astype(jnp.int32)
    emb = _edge_emb(edge_attr, lin_edge_W.T, lin_edge_b.reshape(1, D))
    aggr = _sc_aggregate(src, dst, emb, h)
    return _epilogue(h, aggr, mlp_W1.T, mlp_b1, mlp_W2.T, mlp_b2, eps,
                     bn_gamma, bn_beta)
